# baseline (speedup 1.0000x reference)
import functools

import jax
import jax.numpy as jnp
from jax import lax
from jax.experimental import pallas as pl
from jax.experimental.pallas import tpu as pltpu
from jax.experimental.pallas import tpu_sc as plsc

C = 5
D = 101
EPS = 1e-06
WMSE = 1.0
WMID = 1.0
GAMMA = 5.0
TAUGATE = 0.05
GATETEMP = 0.11

N = 16384
NC, NS, L = 2, 16, 16
NW = NC * NS
EPW = N // NW
CHUNKS = EPW // L
SEC = 512
TBL = 3 * SEC


DCH = (D + L - 1) // L


def _sc_body(score_hbm, y_hbm, ids_hbm, alpha_hbm, part_hbm,
             s_v, y_v, id_v, al_v, en_v, acc):
    wid = lax.axis_index("s") * NC + lax.axis_index("c")
    base = wid * EPW
    pltpu.sync_copy(score_hbm.at[pl.ds(base, EPW)], s_v)
    pltpu.sync_copy(y_hbm.at[pl.ds(base, EPW)], y_v)
    pltpu.sync_copy(ids_hbm.at[pl.ds(base, EPW)], id_v)
    pltpu.sync_copy(alpha_hbm, al_v)

    lane = lax.broadcasted_iota(jnp.int32, (L,), 0)

    @plsc.parallel_loop(0, DCH, unroll=2)
    def _theta_loop(c):
        d = jnp.minimum(c * L + lane, D - 1)
        dflat = d * (C - 1)
        e = jnp.exp(-plsc.load_gather(al_v, [dflat]))
        plsc.store_scatter(en_v, [dflat], e)
        for j in range(1, C - 1):
            x = plsc.load_gather(al_v, [dflat + j])
            g = jnp.exp(-jnp.maximum(x, 0.0)) / (1.0 + jnp.exp(-jnp.abs(x)))
            e = e * g
            plsc.store_scatter(en_v, [dflat + j], e)

    zeros = jnp.zeros((L,), jnp.float32)

    @plsc.parallel_loop(0, TBL // L, unroll=4)
    def _zero_loop(i):
        acc[pl.ds(i * L, L)] = zeros

    inv_temp = 1.0 / GATETEMP

    def body(c, mse_acc):
        off = c * L
        s = s_v[pl.ds(off, L)]
        yv = y_v[pl.ds(off, L)]
        d = id_v[pl.ds(off, L)]
        d = jnp.minimum(jnp.maximum(d, 0), D - 1)
        es = jnp.exp(s)
        dflat = d * (C - 1)
        F = []
        for j in range(C - 1):
            ej = plsc.load_gather(en_v, [dflat + j])
            F.append(1.0 / (1.0 + es * ej))
        p = [F[0]] + [F[j] - F[j - 1] for j in range(1, C - 1)] + [1.0 - F[C - 2]]
        p = [jnp.maximum(q, 0.0) for q in p]
        psum = p[0] + p[1] + p[2] + p[3] + p[4]
        rinv = 1.0 / jnp.maximum(psum, EPS)
        pe = []
        for q in p:
            q = q * rinv
            g = 1.0 / (1.0 + jnp.exp((TAUGATE - q) * inv_temp))
            z = jnp.maximum(q * g, EPS)
            z2 = z * z
            pe.append(z2 * z2 * z)
        esum = pe[0] + pe[1] + pe[2] + pe[3] + pe[4]
        r2 = 1.0 / jnp.maximum(esum, EPS)
        idx0 = d * C
        for k in range(C):
            w = pe[k] * r2
            ws = w * s
            idx = idx0 + k
            plsc.addupdate_scatter(acc, [idx], w)
            plsc.addupdate_scatter(acc, [idx + SEC], ws)
            plsc.addupdate_scatter(acc, [idx + 2 * SEC], ws * s)
        dd = s - yv
        return mse_acc + dd * dd

    mse = plsc.parallel_loop(
        0, CHUNKS, unroll=4, carry=jnp.zeros((L,), jnp.float32)
    )(functools.partial(body))
    sec = (lane >= 7).astype(jnp.int32) + (lane >= 14).astype(jnp.int32)
    plsc.addupdate_scatter(acc, [505 + lane + sec * (SEC - 7)], mse)

    pltpu.sync_copy(acc, part_hbm.at[wid])


_sc_call = pl.kernel(
    _sc_body,
    out_type=jax.ShapeDtypeStruct((NW, TBL), jnp.float32),
    mesh=plsc.VectorSubcoreMesh(
        core_axis_name="c", subcore_axis_name="s",
        num_cores=NC, num_subcores=NS,
    ),
    compiler_params=pltpu.CompilerParams(needs_layout_passes=False),
    scratch_types=[
        pltpu.VMEM((EPW,), jnp.float32),
        pltpu.VMEM((EPW,), jnp.float32),
        pltpu.VMEM((EPW,), jnp.int32),
        pltpu.VMEM((D * (C - 1),), jnp.float32),
        pltpu.VMEM((D * (C - 1),), jnp.float32),
        pltpu.VMEM((TBL,), jnp.float32),
    ],
)


def _final_body(part_ref, out_ref):
    t = jnp.sum(part_ref[...], axis=0)
    t3 = jnp.reshape(t, (3, SEC))
    valid = (lax.broadcasted_iota(jnp.int32, (1, SEC), 1) < D * C)
    m = jnp.where(valid, t3[0:1, :], 0.0)
    sw = jnp.where(valid, t3[1:2, :], 0.0)
    s2 = jnp.where(valid, t3[2:3, :], 0.0)
    mu = sw / jnp.maximum(m, EPS)
    num = s2 - 2.0 * mu * sw + mu * mu * m
    lmid = jnp.sum(num) / jnp.maximum(jnp.sum(m), EPS)
    mse = jnp.sum(jnp.where(valid, 0.0, t3)) / N
    out_ref[...] = jnp.reshape(WMSE * mse + WMID * lmid, (1, 1))


_final_call = pl.pallas_call(
    _final_body,
    out_shape=jax.ShapeDtypeStruct((1, 1), jnp.float32),
)


def kernel(score, y_cont, departement_ids, alpha):
    s = score.reshape(-1).astype(jnp.float32)
    y = y_cont.reshape(-1).astype(jnp.float32)
    ids = departement_ids.reshape(-1).astype(jnp.int32)
    al = alpha.reshape(-1).astype(jnp.float32)
    part = _sc_call(s, y, ids, al)
    return _final_call(part)[0, 0]


# speedup vs baseline: 1.0448x; 1.0448x over previous
import functools

import jax
import jax.numpy as jnp
from jax import lax
from jax.experimental import pallas as pl
from jax.experimental.pallas import tpu as pltpu
from jax.experimental.pallas import tpu_sc as plsc

C = 5
D = 101
EPS = 1e-06
WMSE = 1.0
WMID = 1.0
GAMMA = 5.0
TAUGATE = 0.05
GATETEMP = 0.11

N = 16384
NC, NS, L = 2, 16, 16
NW = NC * NS
EPW = N // NW
CHUNKS = EPW // L
SEC = 512
TBL = 3 * SEC


DCH = (D + L - 1) // L


def _sc_body(score_hbm, y_hbm, ids_hbm, alpha_hbm, part_hbm,
             s_v, y_v, id_v, al_v, en_v, acc):
    wid = lax.axis_index("s") * NC + lax.axis_index("c")
    base = wid * EPW
    pltpu.sync_copy(score_hbm.at[pl.ds(base, EPW)], s_v)
    pltpu.sync_copy(y_hbm.at[pl.ds(base, EPW)], y_v)
    pltpu.sync_copy(ids_hbm.at[pl.ds(base, EPW)], id_v)
    pltpu.sync_copy(alpha_hbm, al_v)

    lane = lax.broadcasted_iota(jnp.int32, (L,), 0)

    @plsc.parallel_loop(0, DCH, unroll=2)
    def _theta_loop(c):
        d = jnp.minimum(c * L + lane, D - 1)
        dflat = d * (C - 1)
        e = jnp.exp(-plsc.load_gather(al_v, [dflat]))
        plsc.store_scatter(en_v, [dflat], e)
        for j in range(1, C - 1):
            x = plsc.load_gather(al_v, [dflat + j])
            g = jnp.exp(-jnp.maximum(x, 0.0)) / (1.0 + jnp.exp(-jnp.abs(x)))
            e = e * g
            plsc.store_scatter(en_v, [dflat + j], e)

    zeros = jnp.zeros((L,), jnp.float32)

    @plsc.parallel_loop(0, TBL // L, unroll=4)
    def _zero_loop(i):
        acc[pl.ds(i * L, L)] = zeros

    inv_temp = 1.0 / GATETEMP

    def body(c, mse_acc):
        off = c * L
        s = s_v[pl.ds(off, L)]
        yv = y_v[pl.ds(off, L)]
        d = id_v[pl.ds(off, L)]
        d = jnp.minimum(jnp.maximum(d, 0), D - 1)
        es = jnp.exp(s)
        dflat = d * (C - 1)
        x = [es * plsc.load_gather(en_v, [dflat + j]) for j in range(C - 1)]
        cd = [1.0 + xj for xj in x]
        c01 = cd[0] * cd[1]
        c23 = cd[2] * cd[3]
        c123 = cd[1] * c23
        c012 = c01 * cd[2]
        c03 = cd[0] * cd[3]
        Q = cd[0] * c123
        p = [c123,
             (x[0] - x[1]) * c23,
             (x[1] - x[2]) * c03,
             (x[2] - x[3]) * c01,
             x[3] * c012]
        p = [jnp.maximum(v, 0.0) for v in p]
        psum = p[0] + p[1] + p[2] + p[3] + p[4]
        rinv = 1.0 / jnp.maximum(psum, EPS * Q)
        qs = [v * rinv for v in p]
        a = [1.0 + jnp.exp((TAUGATE - q) * inv_temp) for q in qs]
        pre1 = a[0] * a[1]
        pre2 = pre1 * a[2]
        suf2 = a[3] * a[4]
        suf1 = a[2] * suf2
        suf0 = a[1] * suf1
        b = [suf0, a[0] * suf1, pre1 * suf2, pre2 * a[4], pre2 * a[3]]
        P = a[0] * suf0
        epsP = EPS * P
        P2 = P * P
        P5 = P2 * P2 * P
        pe = []
        for q, bk in zip(qs, b):
            z = jnp.maximum(q * bk, epsP)
            z2 = z * z
            pe.append(z2 * z2 * z)
        esum = pe[0] + pe[1] + pe[2] + pe[3] + pe[4]
        r2 = 1.0 / jnp.maximum(esum, EPS * P5)
        idx0 = d * C
        for k in range(C):
            w = pe[k] * r2
            ws = w * s
            idx = idx0 + k
            plsc.addupdate_scatter(acc, [idx], w)
            plsc.addupdate_scatter(acc, [idx + SEC], ws)
            plsc.addupdate_scatter(acc, [idx + 2 * SEC], ws * s)
        dd = s - yv
        return mse_acc + dd * dd

    mse = plsc.parallel_loop(
        0, CHUNKS, unroll=2, carry=jnp.zeros((L,), jnp.float32)
    )(functools.partial(body))
    sec = (lane >= 7).astype(jnp.int32) + (lane >= 14).astype(jnp.int32)
    plsc.addupdate_scatter(acc, [505 + lane + sec * (SEC - 7)], mse)

    pltpu.sync_copy(acc, part_hbm.at[wid])


_sc_call = pl.kernel(
    _sc_body,
    out_type=jax.ShapeDtypeStruct((NW, TBL), jnp.float32),
    mesh=plsc.VectorSubcoreMesh(
        core_axis_name="c", subcore_axis_name="s",
        num_cores=NC, num_subcores=NS,
    ),
    compiler_params=pltpu.CompilerParams(needs_layout_passes=False),
    scratch_types=[
        pltpu.VMEM((EPW,), jnp.float32),
        pltpu.VMEM((EPW,), jnp.float32),
        pltpu.VMEM((EPW,), jnp.int32),
        pltpu.VMEM((D * (C - 1),), jnp.float32),
        pltpu.VMEM((D * (C - 1),), jnp.float32),
        pltpu.VMEM((TBL,), jnp.float32),
    ],
)


def _final_body(part_ref, out_ref):
    t = jnp.sum(part_ref[...], axis=0)
    t3 = jnp.reshape(t, (3, SEC))
    valid = (lax.broadcasted_iota(jnp.int32, (1, SEC), 1) < D * C)
    m = jnp.where(valid, t3[0:1, :], 0.0)
    sw = jnp.where(valid, t3[1:2, :], 0.0)
    s2 = jnp.where(valid, t3[2:3, :], 0.0)
    mu = sw / jnp.maximum(m, EPS)
    num = s2 - 2.0 * mu * sw + mu * mu * m
    lmid = jnp.sum(num) / jnp.maximum(jnp.sum(m), EPS)
    mse = jnp.sum(jnp.where(valid, 0.0, t3)) / N
    out_ref[...] = jnp.reshape(WMSE * mse + WMID * lmid, (1, 1))


_final_call = pl.pallas_call(
    _final_body,
    out_shape=jax.ShapeDtypeStruct((1, 1), jnp.float32),
)


def kernel(score, y_cont, departement_ids, alpha):
    s = score.reshape(-1).astype(jnp.float32)
    y = y_cont.reshape(-1).astype(jnp.float32)
    ids = departement_ids.reshape(-1).astype(jnp.int32)
    al = alpha.reshape(-1).astype(jnp.float32)
    part = _sc_call(s, y, ids, al)
    return _final_call(part)[0, 0]


# speedup vs baseline: 1.1225x; 1.0744x over previous
import functools

import jax
import jax.numpy as jnp
from jax import lax
from jax.experimental import pallas as pl
from jax.experimental.pallas import tpu as pltpu
from jax.experimental.pallas import tpu_sc as plsc

C = 5
D = 101
EPS = 1e-06
WMSE = 1.0
WMID = 1.0
GAMMA = 5.0
TAUGATE = 0.05
GATETEMP = 0.11

N = 16384
NC, NS, L = 2, 16, 16
NW = NC * NS
EPW = N // NW
CHUNKS = EPW // L
SEC = 512
TBL = 3 * SEC


DCH = (D + L - 1) // L


def _sc_body(score_hbm, y_hbm, ids_hbm, alpha_hbm, part_hbm,
             s_v, y_v, id_v, al_v, en_v, acc, asem, dsem):
    wid = lax.axis_index("s") * NC + lax.axis_index("c")
    base = wid * EPW
    acp = pltpu.async_copy(alpha_hbm, al_v, asem)
    scp = pltpu.async_copy(score_hbm.at[pl.ds(base, EPW)], s_v, dsem)
    ycp = pltpu.async_copy(y_hbm.at[pl.ds(base, EPW)], y_v, dsem)
    icp = pltpu.async_copy(ids_hbm.at[pl.ds(base, EPW)], id_v, dsem)

    zeros = jnp.zeros((L,), jnp.float32)

    @plsc.parallel_loop(0, TBL // L, unroll=4)
    def _zero_loop(i):
        acc[pl.ds(i * L, L)] = zeros

    acp.wait()

    lane = lax.broadcasted_iota(jnp.int32, (L,), 0)

    @plsc.parallel_loop(0, DCH, unroll=2)
    def _theta_loop(c):
        d = jnp.minimum(c * L + lane, D - 1)
        dflat = d * (C - 1)
        e = jnp.exp(-plsc.load_gather(al_v, [dflat]))
        plsc.store_scatter(en_v, [dflat], e)
        for j in range(1, C - 1):
            x = plsc.load_gather(al_v, [dflat + j])
            g = jnp.exp(-jnp.maximum(x, 0.0)) / (1.0 + jnp.exp(-jnp.abs(x)))
            e = e * g
            plsc.store_scatter(en_v, [dflat + j], e)

    scp.wait()
    ycp.wait()
    icp.wait()

    inv_temp = 1.0 / GATETEMP

    def body(c, mse_acc):
        off = c * L
        s = s_v[pl.ds(off, L)]
        yv = y_v[pl.ds(off, L)]
        d = id_v[pl.ds(off, L)]
        d = jnp.minimum(jnp.maximum(d, 0), D - 1)
        es = jnp.exp(s)
        dflat = d * (C - 1)
        F = []
        for j in range(C - 1):
            ej = plsc.load_gather(en_v, [dflat + j])
            F.append(1.0 / (1.0 + es * ej))
        p = [F[0]] + [F[j] - F[j - 1] for j in range(1, C - 1)] + [1.0 - F[C - 2]]
        p = [jnp.maximum(q, 0.0) for q in p]
        psum = p[0] + p[1] + p[2] + p[3] + p[4]
        rinv = 1.0 / jnp.maximum(psum, EPS)
        pe = []
        for q in p:
            q = q * rinv
            g = 1.0 / (1.0 + jnp.exp((TAUGATE - q) * inv_temp))
            z = jnp.maximum(q * g, EPS)
            z2 = z * z
            pe.append(z2 * z2 * z)
        esum = pe[0] + pe[1] + pe[2] + pe[3] + pe[4]
        r2 = 1.0 / jnp.maximum(esum, EPS)
        idx0 = d * C
        for k in range(C):
            w = pe[k] * r2
            ws = w * s
            idx = idx0 + k
            plsc.addupdate_scatter(acc, [idx], w)
            plsc.addupdate_scatter(acc, [idx + SEC], ws)
            plsc.addupdate_scatter(acc, [idx + 2 * SEC], ws * s)
        dd = s - yv
        return mse_acc + dd * dd

    mse = plsc.parallel_loop(
        0, CHUNKS, unroll=2, carry=jnp.zeros((L,), jnp.float32)
    )(functools.partial(body))
    sec = (lane >= 7).astype(jnp.int32) + (lane >= 14).astype(jnp.int32)
    plsc.addupdate_scatter(acc, [505 + lane + sec * (SEC - 7)], mse)

    pltpu.sync_copy(acc, part_hbm.at[wid])


_sc_call = pl.kernel(
    _sc_body,
    out_type=jax.ShapeDtypeStruct((NW, TBL), jnp.float32),
    mesh=plsc.VectorSubcoreMesh(
        core_axis_name="c", subcore_axis_name="s",
        num_cores=NC, num_subcores=NS,
    ),
    compiler_params=pltpu.CompilerParams(needs_layout_passes=False),
    scratch_types=[
        pltpu.VMEM((EPW,), jnp.float32),
        pltpu.VMEM((EPW,), jnp.float32),
        pltpu.VMEM((EPW,), jnp.int32),
        pltpu.VMEM((D * (C - 1),), jnp.float32),
        pltpu.VMEM((D * (C - 1),), jnp.float32),
        pltpu.VMEM((TBL,), jnp.float32),
        pltpu.SemaphoreType.DMA,
        pltpu.SemaphoreType.DMA,
    ],
)


def _final_body(part_ref, out_ref):
    t = jnp.sum(part_ref[...], axis=0)
    t3 = jnp.reshape(t, (3, SEC))
    valid = (lax.broadcasted_iota(jnp.int32, (1, SEC), 1) < D * C)
    m = jnp.where(valid, t3[0:1, :], 0.0)
    sw = jnp.where(valid, t3[1:2, :], 0.0)
    s2 = jnp.where(valid, t3[2:3, :], 0.0)
    mu = sw / jnp.maximum(m, EPS)
    num = s2 - 2.0 * mu * sw + mu * mu * m
    lmid = jnp.sum(num) / jnp.maximum(jnp.sum(m), EPS)
    mse = jnp.sum(jnp.where(valid, 0.0, t3)) / N
    out_ref[...] = jnp.reshape(WMSE * mse + WMID * lmid, (1, 1))


_final_call = pl.pallas_call(
    _final_body,
    out_shape=jax.ShapeDtypeStruct((1, 1), jnp.float32),
)


def kernel(score, y_cont, departement_ids, alpha):
    s = score.reshape(-1).astype(jnp.float32)
    y = y_cont.reshape(-1).astype(jnp.float32)
    ids = departement_ids.reshape(-1).astype(jnp.int32)
    al = alpha.reshape(-1).astype(jnp.float32)
    part = _sc_call(s, y, ids, al)
    return _final_call(part)[0, 0]
